# baseline (speedup 1.0000x reference)
import functools

import jax
import jax.numpy as jnp
from jax import lax
from jax.experimental import pallas as pl
from jax.experimental.pallas import tpu as pltpu
from jax.experimental.pallas import tpu_sc as plsc

N = 10000
E = 320000
D = 128
LANES = 16
NC = 2
NS = 16
EB = 128
NVR = D // LANES
BLOCKS_PER_CORE = E // (NC * EB)
NP = 10240
ROWS_PER_TILE = NP // NS
ZCH = 128

_f32 = jnp.float32
_i32 = jnp.int32


def _sc_mesh():
    return plsc.VectorSubcoreMesh(
        core_axis_name="c", subcore_axis_name="s", num_cores=NC, num_subcores=NS
    )


def _edge_block_compute(attr_v, rows_v, we_v, be_v):
    @plsc.parallel_loop(0, EB, unroll=8)
    def edge_body(e):
        ev = jnp.full((LANES,), e * LANES, _i32)
        ab = [plsc.load_gather(attr_v, [ev + k]) for k in range(6)]
        for v in range(NVR):
            sl = pl.ds(v * LANES, LANES)
            acc = rows_v[e, sl] + be_v[sl]
            for k in range(6):
                acc = acc + ab[k] * we_v[k, sl]
            rows_v[e, sl] = jnp.maximum(acc, 0.0)


def _make_sc_aggr():
    scratch = [
        pltpu.VMEM_SHARED((NP, D), _f32),
        pltpu.VMEM((EB,), _i32),
        pltpu.VMEM((EB,), _i32),
        pltpu.VMEM((EB,), _i32),
        pltpu.VMEM((EB,), _i32),
        pltpu.VMEM((EB * LANES,), _f32),
        pltpu.VMEM((EB * LANES,), _f32),
        pltpu.VMEM((EB, D), _f32),
        pltpu.VMEM((EB, D), _f32),
        pltpu.VMEM((6, D), _f32),
        pltpu.VMEM((D,), _f32),
        pltpu.SemaphoreType.DMA,
        pltpu.SemaphoreType.DMA,
    ]

    def body(h_hbm, src_hbm, dst_hbm, attr_hbm, we_hbm, be_hbm,
             p_out, aggr_sh, src_v0, src_v1, dst_v0, dst_v1,
             attr_v0, attr_v1, rows_v0, rows_v1, we_v, be_v,
             sem0, sem1):
        c = lax.axis_index("c")
        s = lax.axis_index("s")
        sems = (sem0, sem1)
        src_b = (src_v0, src_v1)
        dst_b = (dst_v0, dst_v1)
        attr_b = (attr_v0, attr_v1)
        rows_b = (rows_v0, rows_v1)

        pltpu.sync_copy(we_hbm, we_v)
        pltpu.sync_copy(be_hbm, be_v)

        def zrow(j, carry):
            for v in range(NVR):
                rows_v0[j, pl.ds(v * LANES, LANES)] = jnp.zeros((LANES,), _f32)
            return carry

        lax.fori_loop(0, EB, zrow, 0)
        for q in range(ROWS_PER_TILE // ZCH):
            r0 = s * ROWS_PER_TILE + q * ZCH
            pltpu.sync_copy(rows_v0.at[pl.ds(0, ZCH), :],
                            aggr_sh.at[pl.ds(r0, ZCH), :])
        plsc.subcore_barrier()

        def load_idx(j, buf):
            base = c * (E // NC) + (s + NS * j) * EB
            pltpu.sync_copy(src_hbm.at[pl.ds(base, EB)], src_b[buf])
            pltpu.sync_copy(dst_hbm.at[pl.ds(base, EB)], dst_b[buf])
            pltpu.sync_copy(attr_hbm.at[pl.ds(base * LANES, EB * LANES)],
                            attr_b[buf])

        def gather_start(buf):
            pltpu.async_copy(h_hbm.at[src_b[buf]], rows_b[buf], sems[buf])

        def gather_wait(buf):
            pltpu.make_async_copy(h_hbm.at[src_b[buf]], rows_b[buf],
                                  sems[buf]).wait()

        def finish(buf):
            gather_wait(buf)
            _edge_block_compute(attr_b[buf], rows_b[buf], we_v, be_v)
            pltpu.sync_copy(rows_b[buf], aggr_sh.at[dst_b[buf]], add=True)

        load_idx(0, 0)
        gather_start(0)

        def body2(t, carry):
            load_idx(2 * t + 1, 1)
            gather_start(1)
            finish(0)

            @pl.when(t < 38)
            def _():
                load_idx(2 * t + 2, 0)
                gather_start(0)

            finish(1)
            return carry

        lax.fori_loop(0, 39, body2, 0)

        @pl.when(s < BLOCKS_PER_CORE - 78 * NS)
        def _():
            load_idx(78, 0)
            gather_start(0)
            finish(0)

        plsc.subcore_barrier()

        r0 = s * ROWS_PER_TILE
        pltpu.sync_copy(
            aggr_sh.at[pl.ds(r0, ROWS_PER_TILE), :],
            p_out.at[c, pl.ds(r0, ROWS_PER_TILE), :],
        )

    return pl.kernel(
        body,
        out_type=[jax.ShapeDtypeStruct((NC, NP, D), _f32)],
        mesh=_sc_mesh(),
        scratch_types=scratch,
        compiler_params=pltpu.CompilerParams(needs_layout_passes=False),
    )


def _make_sc_cnt():
    scratch = [
        pltpu.VMEM_SHARED((NP, D), _f32),
        pltpu.VMEM((EB,), _i32),
        pltpu.VMEM((EB, D), _f32),
        pltpu.SemaphoreType.DMA,
    ]

    def body(dst_hbm, cnt_out, cnt_sh, dst_v, rows_v, sem):
        c = lax.axis_index("c")
        s = lax.axis_index("s")

        def zrow(j, carry):
            for v in range(NVR):
                rows_v[j, pl.ds(v * LANES, LANES)] = jnp.zeros((LANES,), _f32)
            return carry

        lax.fori_loop(0, EB, zrow, 0)
        for q in range(ROWS_PER_TILE // ZCH):
            r0 = s * ROWS_PER_TILE + q * ZCH
            pltpu.sync_copy(rows_v.at[pl.ds(0, ZCH), :], cnt_sh.at[pl.ds(r0, ZCH), :])
        plsc.subcore_barrier()

        def orow(j, carry):
            rows_v[j, pl.ds(0, LANES)] = jnp.ones((LANES,), _f32)
            return carry

        lax.fori_loop(0, EB, orow, 0)

        nblk = 78 + jnp.where(s < BLOCKS_PER_CORE - 78 * NS, 1, 0)

        def blk_body(j, carry):
            b = s + NS * j
            base = c * (E // NC) + b * EB
            pltpu.sync_copy(dst_hbm.at[pl.ds(base, EB)], dst_v)
            pltpu.sync_copy(rows_v, cnt_sh.at[dst_v], add=True)
            return carry

        lax.fori_loop(0, nblk, blk_body, 0)
        plsc.subcore_barrier()

        r0 = s * ROWS_PER_TILE
        pltpu.sync_copy(
            cnt_sh.at[pl.ds(r0, ROWS_PER_TILE), :],
            cnt_out.at[c, pl.ds(r0, ROWS_PER_TILE), :],
        )

    return pl.kernel(
        body,
        out_type=[jax.ShapeDtypeStruct((NC, NP, D), _f32)],
        mesh=_sc_mesh(),
        scratch_types=scratch,
        compiler_params=pltpu.CompilerParams(needs_layout_passes=False),
    )


def _tc_mlp_body(h_ref, p0_ref, p1_ref, c0_ref, c1_ref, scale_ref,
                 w1_ref, b1_ref, w2_ref, b2_ref, out_ref):
    cnt = c0_ref[:, 0:1] + c1_ref[:, 0:1]
    denom = jnp.maximum(cnt, 1.0)
    aggr = (p0_ref[...] + p1_ref[...]) / denom
    hmid = h_ref[...] * scale_ref[...] + aggr
    t = jnp.maximum(
        jnp.dot(hmid, w1_ref[...], preferred_element_type=_f32) + b1_ref[...], 0.0
    )
    o = jnp.dot(t, w2_ref[...], preferred_element_type=_f32) + b2_ref[...]
    out_ref[...] = jnp.maximum(o, 0.0)


def _tc_mlp(h, p0, p1, c0, c1, scale, w1, b1, w2, b2):
    R = 2000
    nb = N // R
    return pl.pallas_call(
        _tc_mlp_body,
        grid=(nb,),
        in_specs=[
            pl.BlockSpec((R, D), lambda i: (i, 0)),
            pl.BlockSpec((R, D), lambda i: (i, 0)),
            pl.BlockSpec((R, D), lambda i: (i, 0)),
            pl.BlockSpec((R, LANES), lambda i: (i, 0)),
            pl.BlockSpec((R, LANES), lambda i: (i, 0)),
            pl.BlockSpec((1, D), lambda i: (0, 0)),
            pl.BlockSpec((D, 2 * D), lambda i: (0, 0)),
            pl.BlockSpec((1, 2 * D), lambda i: (0, 0)),
            pl.BlockSpec((2 * D, D), lambda i: (0, 0)),
            pl.BlockSpec((1, D), lambda i: (0, 0)),
        ],
        out_specs=pl.BlockSpec((R, D), lambda i: (i, 0)),
        out_shape=jax.ShapeDtypeStruct((N, D), _f32),
    )(h, p0, p1, c0, c1, scale, w1, b1, w2, b2)


_sc_aggr = _make_sc_aggr()
_sc_cnt = _make_sc_cnt()


def kernel(x, edge_index, edge_attr, batch, We, be, W1, b1, W2, b2, eps):
    src = edge_index[0]
    dst = edge_index[1]
    attr16 = jnp.pad(edge_attr, ((0, 0), (0, LANES - edge_attr.shape[1]))).reshape(-1)
    h = x
    outs = []
    (cnt,) = _sc_cnt(dst)
    c0, c1 = cnt[0, :N, :LANES], cnt[1, :N, :LANES]
    for i in range(We.shape[0]):
        (p,) = _sc_aggr(h, src, dst, attr16, We[i], be[i])
        scale = jnp.full((1, D), 1.0, _f32) + eps[i]
        h = _tc_mlp(h, p[0, :N], p[1, :N], c0, c1, scale,
                    W1[i], b1[i].reshape(1, -1), W2[i], b2[i].reshape(1, -1))
        outs.append(h)
    return jnp.concatenate(outs, axis=1)


# speedup vs baseline: 1.1839x; 1.1839x over previous
import functools

import jax
import jax.numpy as jnp
from jax import lax
from jax.experimental import pallas as pl
from jax.experimental.pallas import tpu as pltpu
from jax.experimental.pallas import tpu_sc as plsc

N = 10000
E = 320000
D = 128
LANES = 16
NC = 2
NS = 16
EB = 128
NVR = D // LANES
BLOCKS_PER_CORE = E // (NC * EB)
NP = 10240
ROWS_PER_TILE = NP // NS
ZCH = 128

_f32 = jnp.float32
_i32 = jnp.int32


def _sc_mesh():
    return plsc.VectorSubcoreMesh(
        core_axis_name="c", subcore_axis_name="s", num_cores=NC, num_subcores=NS
    )


def _edge_block_compute(attr_v, rows_v, we_v, be_v):
    @plsc.parallel_loop(0, EB, unroll=2)
    def edge_body(e):
        ev = jnp.full((LANES,), e * LANES, _i32)
        ab = [plsc.load_gather(attr_v, [ev + k]) for k in range(6)]
        for v in range(NVR):
            sl = pl.ds(v * LANES, LANES)
            acc = rows_v[e, sl] + be_v[sl]
            for k in range(6):
                acc = acc + ab[k] * we_v[k, sl]
            rows_v[e, sl] = jnp.maximum(acc, 0.0)


def _make_sc_aggr():
    scratch = [
        pltpu.VMEM_SHARED((NP, D), _f32),
        pltpu.VMEM((EB,), _i32),
        pltpu.VMEM((EB,), _i32),
        pltpu.VMEM((EB,), _i32),
        pltpu.VMEM((EB,), _i32),
        pltpu.VMEM((EB * LANES,), _f32),
        pltpu.VMEM((EB * LANES,), _f32),
        pltpu.VMEM((EB, D), _f32),
        pltpu.VMEM((EB, D), _f32),
        pltpu.VMEM((6, D), _f32),
        pltpu.VMEM((D,), _f32),
        pltpu.SemaphoreType.DMA,
        pltpu.SemaphoreType.DMA,
    ]

    def body(h_hbm, src_hbm, dst_hbm, attr_hbm, we_hbm, be_hbm,
             p_out, aggr_sh, src_v0, src_v1, dst_v0, dst_v1,
             attr_v0, attr_v1, rows_v0, rows_v1, we_v, be_v,
             sem0, sem1):
        c = lax.axis_index("c")
        s = lax.axis_index("s")
        sems = (sem0, sem1)
        src_b = (src_v0, src_v1)
        dst_b = (dst_v0, dst_v1)
        attr_b = (attr_v0, attr_v1)
        rows_b = (rows_v0, rows_v1)

        pltpu.sync_copy(we_hbm, we_v)
        pltpu.sync_copy(be_hbm, be_v)

        def zrow(j, carry):
            for v in range(NVR):
                rows_v0[j, pl.ds(v * LANES, LANES)] = jnp.zeros((LANES,), _f32)
            return carry

        lax.fori_loop(0, EB, zrow, 0)
        for q in range(ROWS_PER_TILE // ZCH):
            r0 = s * ROWS_PER_TILE + q * ZCH
            pltpu.sync_copy(rows_v0.at[pl.ds(0, ZCH), :],
                            aggr_sh.at[pl.ds(r0, ZCH), :])
        plsc.subcore_barrier()

        def load_idx(j, buf):
            base = c * (E // NC) + (s + NS * j) * EB
            pltpu.sync_copy(src_hbm.at[pl.ds(base, EB)], src_b[buf])
            pltpu.sync_copy(dst_hbm.at[pl.ds(base, EB)], dst_b[buf])
            pltpu.sync_copy(attr_hbm.at[pl.ds(base * LANES, EB * LANES)],
                            attr_b[buf])

        def gather_start(buf):
            pltpu.async_copy(h_hbm.at[src_b[buf]], rows_b[buf], sems[buf])

        def gather_wait(buf):
            pltpu.make_async_copy(h_hbm.at[src_b[buf]], rows_b[buf],
                                  sems[buf]).wait()

        def finish(buf):
            gather_wait(buf)
            _edge_block_compute(attr_b[buf], rows_b[buf], we_v, be_v)
            pltpu.sync_copy(rows_b[buf], aggr_sh.at[dst_b[buf]], add=True)

        load_idx(0, 0)
        gather_start(0)

        def body2(t, carry):
            load_idx(2 * t + 1, 1)
            gather_start(1)
            finish(0)

            @pl.when(t < 38)
            def _():
                load_idx(2 * t + 2, 0)
                gather_start(0)

            finish(1)
            return carry

        lax.fori_loop(0, 39, body2, 0)

        @pl.when(s < BLOCKS_PER_CORE - 78 * NS)
        def _():
            load_idx(78, 0)
            gather_start(0)
            finish(0)

        plsc.subcore_barrier()

        r0 = s * ROWS_PER_TILE
        pltpu.sync_copy(
            aggr_sh.at[pl.ds(r0, ROWS_PER_TILE), :],
            p_out.at[c, pl.ds(r0, ROWS_PER_TILE), :],
        )

    return pl.kernel(
        body,
        out_type=[jax.ShapeDtypeStruct((NC, NP, D), _f32)],
        mesh=_sc_mesh(),
        scratch_types=scratch,
        compiler_params=pltpu.CompilerParams(needs_layout_passes=False),
    )


def _make_sc_cnt():
    scratch = [
        pltpu.VMEM_SHARED((NP, D), _f32),
        pltpu.VMEM((EB,), _i32),
        pltpu.VMEM((EB, D), _f32),
        pltpu.SemaphoreType.DMA,
    ]

    def body(dst_hbm, cnt_out, cnt_sh, dst_v, rows_v, sem):
        c = lax.axis_index("c")
        s = lax.axis_index("s")

        def zrow(j, carry):
            for v in range(NVR):
                rows_v[j, pl.ds(v * LANES, LANES)] = jnp.zeros((LANES,), _f32)
            return carry

        lax.fori_loop(0, EB, zrow, 0)
        for q in range(ROWS_PER_TILE // ZCH):
            r0 = s * ROWS_PER_TILE + q * ZCH
            pltpu.sync_copy(rows_v.at[pl.ds(0, ZCH), :], cnt_sh.at[pl.ds(r0, ZCH), :])
        plsc.subcore_barrier()

        def orow(j, carry):
            rows_v[j, pl.ds(0, LANES)] = jnp.ones((LANES,), _f32)
            return carry

        lax.fori_loop(0, EB, orow, 0)

        nblk = 78 + jnp.where(s < BLOCKS_PER_CORE - 78 * NS, 1, 0)

        def blk_body(j, carry):
            b = s + NS * j
            base = c * (E // NC) + b * EB
            pltpu.sync_copy(dst_hbm.at[pl.ds(base, EB)], dst_v)
            pltpu.sync_copy(rows_v, cnt_sh.at[dst_v], add=True)
            return carry

        lax.fori_loop(0, nblk, blk_body, 0)
        plsc.subcore_barrier()

        r0 = s * ROWS_PER_TILE
        pltpu.sync_copy(
            cnt_sh.at[pl.ds(r0, ROWS_PER_TILE), :],
            cnt_out.at[c, pl.ds(r0, ROWS_PER_TILE), :],
        )

    return pl.kernel(
        body,
        out_type=[jax.ShapeDtypeStruct((NC, NP, D), _f32)],
        mesh=_sc_mesh(),
        scratch_types=scratch,
        compiler_params=pltpu.CompilerParams(needs_layout_passes=False),
    )


def _tc_mlp_body(h_ref, p0_ref, p1_ref, c0_ref, c1_ref, scale_ref,
                 w1_ref, b1_ref, w2_ref, b2_ref, out_ref):
    cnt = c0_ref[:, 0:1] + c1_ref[:, 0:1]
    denom = jnp.maximum(cnt, 1.0)
    aggr = (p0_ref[...] + p1_ref[...]) / denom
    hmid = h_ref[...] * scale_ref[...] + aggr
    t = jnp.maximum(
        jnp.dot(hmid, w1_ref[...], preferred_element_type=_f32) + b1_ref[...], 0.0
    )
    o = jnp.dot(t, w2_ref[...], preferred_element_type=_f32) + b2_ref[...]
    out_ref[...] = jnp.maximum(o, 0.0)


def _tc_mlp(h, p0, p1, c0, c1, scale, w1, b1, w2, b2):
    R = 2000
    nb = N // R
    return pl.pallas_call(
        _tc_mlp_body,
        grid=(nb,),
        in_specs=[
            pl.BlockSpec((R, D), lambda i: (i, 0)),
            pl.BlockSpec((R, D), lambda i: (i, 0)),
            pl.BlockSpec((R, D), lambda i: (i, 0)),
            pl.BlockSpec((R, LANES), lambda i: (i, 0)),
            pl.BlockSpec((R, LANES), lambda i: (i, 0)),
            pl.BlockSpec((1, D), lambda i: (0, 0)),
            pl.BlockSpec((D, 2 * D), lambda i: (0, 0)),
            pl.BlockSpec((1, 2 * D), lambda i: (0, 0)),
            pl.BlockSpec((2 * D, D), lambda i: (0, 0)),
            pl.BlockSpec((1, D), lambda i: (0, 0)),
        ],
        out_specs=pl.BlockSpec((R, D), lambda i: (i, 0)),
        out_shape=jax.ShapeDtypeStruct((N, D), _f32),
    )(h, p0, p1, c0, c1, scale, w1, b1, w2, b2)


_sc_aggr = _make_sc_aggr()
_sc_cnt = _make_sc_cnt()


def kernel(x, edge_index, edge_attr, batch, We, be, W1, b1, W2, b2, eps):
    src = edge_index[0]
    dst = edge_index[1]
    attr16 = jnp.pad(edge_attr, ((0, 0), (0, LANES - edge_attr.shape[1]))).reshape(-1)
    h = x
    outs = []
    (cnt,) = _sc_cnt(dst)
    c0, c1 = cnt[0, :N, :LANES], cnt[1, :N, :LANES]
    for i in range(We.shape[0]):
        (p,) = _sc_aggr(h, src, dst, attr16, We[i], be[i])
        scale = jnp.full((1, D), 1.0, _f32) + eps[i]
        h = _tc_mlp(h, p[0, :N], p[1, :N], c0, c1, scale,
                    W1[i], b1[i].reshape(1, -1), W2[i], b2[i].reshape(1, -1))
        outs.append(h)
    return jnp.concatenate(outs, axis=1)


# speedup vs baseline: 1.2292x; 1.0383x over previous
import functools

import jax
import jax.numpy as jnp
from jax import lax
from jax.experimental import pallas as pl
from jax.experimental.pallas import tpu as pltpu
from jax.experimental.pallas import tpu_sc as plsc

N = 10000
E = 320000
D = 128
LANES = 16
NC = 2
NS = 16
EB = 128
NVR = D // LANES
BLOCKS_PER_CORE = E // (NC * EB)
NP = 10240
ROWS_PER_TILE = NP // NS
ZCH = 128

_f32 = jnp.float32
_i32 = jnp.int32


def _sc_mesh():
    return plsc.VectorSubcoreMesh(
        core_axis_name="c", subcore_axis_name="s", num_cores=NC, num_subcores=NS
    )


_GDN = lax.GatherDimensionNumbers(
    offset_dims=(), collapsed_slice_dims=(0,), start_index_map=(0,))


def _bcast_lane(vec, k):
    idx = jnp.full((LANES, 1), k, _i32)
    return lax.gather(vec, idx, _GDN, (1,),
                      mode=lax.GatherScatterMode.PROMISE_IN_BOUNDS)


def _edge_block_compute(attr_v, rows_v, we_v, be_v):
    @plsc.parallel_loop(0, EB, unroll=2)
    def edge_body(e):
        a = attr_v[pl.ds(e * LANES, LANES)]
        ab = [_bcast_lane(a, k) for k in range(6)]
        for v in range(NVR):
            sl = pl.ds(v * LANES, LANES)
            acc = rows_v[e, sl] + be_v[sl]
            for k in range(6):
                acc = acc + ab[k] * we_v[k, sl]
            rows_v[e, sl] = jnp.maximum(acc, 0.0)


def _make_sc_aggr():
    scratch = [
        pltpu.VMEM_SHARED((NP, D), _f32),
        pltpu.VMEM((EB,), _i32),
        pltpu.VMEM((EB,), _i32),
        pltpu.VMEM((EB,), _i32),
        pltpu.VMEM((EB,), _i32),
        pltpu.VMEM((EB * LANES,), _f32),
        pltpu.VMEM((EB * LANES,), _f32),
        pltpu.VMEM((EB, D), _f32),
        pltpu.VMEM((EB, D), _f32),
        pltpu.VMEM((6, D), _f32),
        pltpu.VMEM((D,), _f32),
        pltpu.SemaphoreType.DMA,
        pltpu.SemaphoreType.DMA,
    ]

    def body(h_hbm, src_hbm, dst_hbm, attr_hbm, we_hbm, be_hbm,
             p_out, aggr_sh, src_v0, src_v1, dst_v0, dst_v1,
             attr_v0, attr_v1, rows_v0, rows_v1, we_v, be_v,
             sem0, sem1):
        c = lax.axis_index("c")
        s = lax.axis_index("s")
        sems = (sem0, sem1)
        src_b = (src_v0, src_v1)
        dst_b = (dst_v0, dst_v1)
        attr_b = (attr_v0, attr_v1)
        rows_b = (rows_v0, rows_v1)

        pltpu.sync_copy(we_hbm, we_v)
        pltpu.sync_copy(be_hbm, be_v)

        def zrow(j, carry):
            for v in range(NVR):
                rows_v0[j, pl.ds(v * LANES, LANES)] = jnp.zeros((LANES,), _f32)
            return carry

        lax.fori_loop(0, EB, zrow, 0)
        for q in range(ROWS_PER_TILE // ZCH):
            r0 = s * ROWS_PER_TILE + q * ZCH
            pltpu.sync_copy(rows_v0.at[pl.ds(0, ZCH), :],
                            aggr_sh.at[pl.ds(r0, ZCH), :])
        plsc.subcore_barrier()

        def load_idx(j, buf):
            base = c * (E // NC) + (s + NS * j) * EB
            pltpu.sync_copy(src_hbm.at[pl.ds(base, EB)], src_b[buf])
            pltpu.sync_copy(dst_hbm.at[pl.ds(base, EB)], dst_b[buf])
            pltpu.sync_copy(attr_hbm.at[pl.ds(base * LANES, EB * LANES)],
                            attr_b[buf])

        def gather_start(buf):
            pltpu.async_copy(h_hbm.at[src_b[buf]], rows_b[buf], sems[buf])

        def gather_wait(buf):
            pltpu.make_async_copy(h_hbm.at[src_b[buf]], rows_b[buf],
                                  sems[buf]).wait()

        def finish(buf):
            gather_wait(buf)
            _edge_block_compute(attr_b[buf], rows_b[buf], we_v, be_v)
            pltpu.sync_copy(rows_b[buf], aggr_sh.at[dst_b[buf]], add=True)

        load_idx(0, 0)
        gather_start(0)

        def body2(t, carry):
            load_idx(2 * t + 1, 1)
            gather_start(1)
            finish(0)

            @pl.when(t < 38)
            def _():
                load_idx(2 * t + 2, 0)
                gather_start(0)

            finish(1)
            return carry

        lax.fori_loop(0, 39, body2, 0)

        @pl.when(s < BLOCKS_PER_CORE - 78 * NS)
        def _():
            load_idx(78, 0)
            gather_start(0)
            finish(0)

        plsc.subcore_barrier()

        r0 = s * ROWS_PER_TILE
        pltpu.sync_copy(
            aggr_sh.at[pl.ds(r0, ROWS_PER_TILE), :],
            p_out.at[c, pl.ds(r0, ROWS_PER_TILE), :],
        )

    return pl.kernel(
        body,
        out_type=[jax.ShapeDtypeStruct((NC, NP, D), _f32)],
        mesh=_sc_mesh(),
        scratch_types=scratch,
        compiler_params=pltpu.CompilerParams(needs_layout_passes=False),
    )


def _make_sc_cnt():
    scratch = [
        pltpu.VMEM_SHARED((NP, D), _f32),
        pltpu.VMEM((EB,), _i32),
        pltpu.VMEM((EB, D), _f32),
        pltpu.SemaphoreType.DMA,
    ]

    def body(dst_hbm, cnt_out, cnt_sh, dst_v, rows_v, sem):
        c = lax.axis_index("c")
        s = lax.axis_index("s")

        def zrow(j, carry):
            for v in range(NVR):
                rows_v[j, pl.ds(v * LANES, LANES)] = jnp.zeros((LANES,), _f32)
            return carry

        lax.fori_loop(0, EB, zrow, 0)
        for q in range(ROWS_PER_TILE // ZCH):
            r0 = s * ROWS_PER_TILE + q * ZCH
            pltpu.sync_copy(rows_v.at[pl.ds(0, ZCH), :], cnt_sh.at[pl.ds(r0, ZCH), :])
        plsc.subcore_barrier()

        def orow(j, carry):
            rows_v[j, pl.ds(0, LANES)] = jnp.ones((LANES,), _f32)
            return carry

        lax.fori_loop(0, EB, orow, 0)

        nblk = 78 + jnp.where(s < BLOCKS_PER_CORE - 78 * NS, 1, 0)

        def blk_body(j, carry):
            b = s + NS * j
            base = c * (E // NC) + b * EB
            pltpu.sync_copy(dst_hbm.at[pl.ds(base, EB)], dst_v)
            pltpu.sync_copy(rows_v, cnt_sh.at[dst_v], add=True)
            return carry

        lax.fori_loop(0, nblk, blk_body, 0)
        plsc.subcore_barrier()

        r0 = s * ROWS_PER_TILE
        pltpu.sync_copy(
            cnt_sh.at[pl.ds(r0, ROWS_PER_TILE), :],
            cnt_out.at[c, pl.ds(r0, ROWS_PER_TILE), :],
        )

    return pl.kernel(
        body,
        out_type=[jax.ShapeDtypeStruct((NC, NP, D), _f32)],
        mesh=_sc_mesh(),
        scratch_types=scratch,
        compiler_params=pltpu.CompilerParams(needs_layout_passes=False),
    )


def _tc_mlp_body(h_ref, p0_ref, p1_ref, c0_ref, c1_ref, scale_ref,
                 w1_ref, b1_ref, w2_ref, b2_ref, out_ref):
    cnt = c0_ref[:, 0:1] + c1_ref[:, 0:1]
    denom = jnp.maximum(cnt, 1.0)
    aggr = (p0_ref[...] + p1_ref[...]) / denom
    hmid = h_ref[...] * scale_ref[...] + aggr
    t = jnp.maximum(
        jnp.dot(hmid, w1_ref[...], preferred_element_type=_f32) + b1_ref[...], 0.0
    )
    o = jnp.dot(t, w2_ref[...], preferred_element_type=_f32) + b2_ref[...]
    out_ref[...] = jnp.maximum(o, 0.0)


def _tc_mlp(h, p0, p1, c0, c1, scale, w1, b1, w2, b2):
    R = 2000
    nb = N // R
    return pl.pallas_call(
        _tc_mlp_body,
        grid=(nb,),
        in_specs=[
            pl.BlockSpec((R, D), lambda i: (i, 0)),
            pl.BlockSpec((R, D), lambda i: (i, 0)),
            pl.BlockSpec((R, D), lambda i: (i, 0)),
            pl.BlockSpec((R, LANES), lambda i: (i, 0)),
            pl.BlockSpec((R, LANES), lambda i: (i, 0)),
            pl.BlockSpec((1, D), lambda i: (0, 0)),
            pl.BlockSpec((D, 2 * D), lambda i: (0, 0)),
            pl.BlockSpec((1, 2 * D), lambda i: (0, 0)),
            pl.BlockSpec((2 * D, D), lambda i: (0, 0)),
            pl.BlockSpec((1, D), lambda i: (0, 0)),
        ],
        out_specs=pl.BlockSpec((R, D), lambda i: (i, 0)),
        out_shape=jax.ShapeDtypeStruct((N, D), _f32),
    )(h, p0, p1, c0, c1, scale, w1, b1, w2, b2)


_sc_aggr = _make_sc_aggr()
_sc_cnt = _make_sc_cnt()


def kernel(x, edge_index, edge_attr, batch, We, be, W1, b1, W2, b2, eps):
    src = edge_index[0]
    dst = edge_index[1]
    attr16 = jnp.pad(edge_attr, ((0, 0), (0, LANES - edge_attr.shape[1]))).reshape(-1)
    h = x
    outs = []
    (cnt,) = _sc_cnt(dst)
    c0, c1 = cnt[0, :N, :LANES], cnt[1, :N, :LANES]
    for i in range(We.shape[0]):
        (p,) = _sc_aggr(h, src, dst, attr16, We[i], be[i])
        scale = jnp.full((1, D), 1.0, _f32) + eps[i]
        h = _tc_mlp(h, p[0, :N], p[1, :N], c0, c1, scale,
                    W1[i], b1[i].reshape(1, -1), W2[i], b2[i].reshape(1, -1))
        outs.append(h)
    return jnp.concatenate(outs, axis=1)


# speedup vs baseline: 1.4295x; 1.1629x over previous
import jax
import jax.numpy as jnp
from jax import lax
from jax.experimental import pallas as pl
from jax.experimental.pallas import tpu as pltpu
from jax.experimental.pallas import tpu_sc as plsc

N = 10000
E = 320000
D = 128
LANES = 16
NC = 2
NS = 16
EB = 128
NVR = D // LANES
BLOCKS_PER_CORE = E // (NC * EB)
NP = 10240
ROWS_PER_TILE = NP // NS
ZCH = 128

_f32 = jnp.float32
_i32 = jnp.int32


def _sc_mesh():
    return plsc.VectorSubcoreMesh(
        core_axis_name="c", subcore_axis_name="s", num_cores=NC, num_subcores=NS
    )


_GDN = lax.GatherDimensionNumbers(
    offset_dims=(), collapsed_slice_dims=(0,), start_index_map=(0,))


def _bcast_lane(vec, k):
    idx = jnp.full((LANES, 1), k, _i32)
    return lax.gather(vec, idx, _GDN, (1,),
                      mode=lax.GatherScatterMode.PROMISE_IN_BOUNDS)


def _edge_block_compute(attr_v, rows_v, we_v, be_v):
    for half in range(4):
        vs = list(range(half * (NVR // 4), (half + 1) * (NVR // 4)))
        wregs = [[we_v[k, pl.ds(v * LANES, LANES)] for k in range(6)] for v in vs]
        bregs = [be_v[pl.ds(v * LANES, LANES)] for v in vs]

        @plsc.parallel_loop(0, EB, unroll=4)
        def edge_body(e):
            a = attr_v[pl.ds(e * LANES, LANES)]
            ab = [_bcast_lane(a, k) for k in range(6)]
            for i, v in enumerate(vs):
                sl = pl.ds(v * LANES, LANES)
                m = [ab[k] * wregs[i][k] for k in range(6)]
                acc = ((rows_v[e, sl] + bregs[i]) + (m[0] + m[1])) + (
                    (m[2] + m[3]) + (m[4] + m[5]))
                rows_v[e, sl] = jnp.maximum(acc, 0.0)


def _make_sc_aggr():
    scratch = [
        pltpu.VMEM_SHARED((NP, D), _f32),
        pltpu.VMEM((EB,), _i32),
        pltpu.VMEM((EB,), _i32),
        pltpu.VMEM((EB,), _i32),
        pltpu.VMEM((EB,), _i32),
        pltpu.VMEM((EB * LANES,), _f32),
        pltpu.VMEM((EB * LANES,), _f32),
        pltpu.VMEM((EB, D), _f32),
        pltpu.VMEM((EB, D), _f32),
        pltpu.VMEM((6, D), _f32),
        pltpu.VMEM((D,), _f32),
        pltpu.SemaphoreType.DMA,
        pltpu.SemaphoreType.DMA,
    ]

    def body(h_hbm, src_hbm, dst_hbm, attr_hbm, we_hbm, be_hbm,
             p_out, aggr_sh, src_v0, src_v1, dst_v0, dst_v1,
             attr_v0, attr_v1, rows_v0, rows_v1, we_v, be_v,
             sem0, sem1):
        c = lax.axis_index("c")
        s = lax.axis_index("s")
        sems = (sem0, sem1)
        src_b = (src_v0, src_v1)
        dst_b = (dst_v0, dst_v1)
        attr_b = (attr_v0, attr_v1)
        rows_b = (rows_v0, rows_v1)

        pltpu.sync_copy(we_hbm, we_v)
        pltpu.sync_copy(be_hbm, be_v)

        def zrow(j, carry):
            for v in range(NVR):
                rows_v0[j, pl.ds(v * LANES, LANES)] = jnp.zeros((LANES,), _f32)
            return carry

        lax.fori_loop(0, EB, zrow, 0)
        for q in range(ROWS_PER_TILE // ZCH):
            r0 = s * ROWS_PER_TILE + q * ZCH
            pltpu.sync_copy(rows_v0.at[pl.ds(0, ZCH), :],
                            aggr_sh.at[pl.ds(r0, ZCH), :])
        plsc.subcore_barrier()

        def load_idx(j, buf):
            base = c * (E // NC) + (s + NS * j) * EB
            pltpu.sync_copy(src_hbm.at[pl.ds(base, EB)], src_b[buf])
            pltpu.sync_copy(dst_hbm.at[pl.ds(base, EB)], dst_b[buf])
            pltpu.sync_copy(attr_hbm.at[pl.ds(base * LANES, EB * LANES)],
                            attr_b[buf])

        def gather_start(buf):
            pltpu.async_copy(h_hbm.at[src_b[buf]], rows_b[buf], sems[buf])

        def gather_wait(buf):
            pltpu.make_async_copy(h_hbm.at[src_b[buf]], rows_b[buf],
                                  sems[buf]).wait()

        def finish(buf):
            gather_wait(buf)
            _edge_block_compute(attr_b[buf], rows_b[buf], we_v, be_v)
            pltpu.sync_copy(rows_b[buf], aggr_sh.at[dst_b[buf]], add=True)

        load_idx(0, 0)
        gather_start(0)

        def body2(t, carry):
            load_idx(2 * t + 1, 1)
            gather_start(1)
            finish(0)

            @pl.when(t < 38)
            def _():
                load_idx(2 * t + 2, 0)
                gather_start(0)

            finish(1)
            return carry

        lax.fori_loop(0, 39, body2, 0)

        @pl.when(s < BLOCKS_PER_CORE - 78 * NS)
        def _():
            load_idx(78, 0)
            gather_start(0)
            finish(0)

        plsc.subcore_barrier()

        r0 = s * ROWS_PER_TILE
        pltpu.sync_copy(
            aggr_sh.at[pl.ds(r0, ROWS_PER_TILE), :],
            p_out.at[c, pl.ds(r0, ROWS_PER_TILE), :],
        )

    return pl.kernel(
        body,
        out_type=[jax.ShapeDtypeStruct((NC, NP, D), _f32)],
        mesh=_sc_mesh(),
        scratch_types=scratch,
        compiler_params=pltpu.CompilerParams(needs_layout_passes=False),
    )


def _make_sc_cnt():
    scratch = [
        pltpu.VMEM_SHARED((NP, D), _f32),
        pltpu.VMEM((EB,), _i32),
        pltpu.VMEM((EB, D), _f32),
        pltpu.SemaphoreType.DMA,
    ]

    def body(dst_hbm, cnt_out, cnt_sh, dst_v, rows_v, sem):
        c = lax.axis_index("c")
        s = lax.axis_index("s")

        def zrow(j, carry):
            for v in range(NVR):
                rows_v[j, pl.ds(v * LANES, LANES)] = jnp.zeros((LANES,), _f32)
            return carry

        lax.fori_loop(0, EB, zrow, 0)
        for q in range(ROWS_PER_TILE // ZCH):
            r0 = s * ROWS_PER_TILE + q * ZCH
            pltpu.sync_copy(rows_v.at[pl.ds(0, ZCH), :], cnt_sh.at[pl.ds(r0, ZCH), :])
        plsc.subcore_barrier()

        def orow(j, carry):
            rows_v[j, pl.ds(0, LANES)] = jnp.ones((LANES,), _f32)
            return carry

        lax.fori_loop(0, EB, orow, 0)

        nblk = 78 + jnp.where(s < BLOCKS_PER_CORE - 78 * NS, 1, 0)

        def blk_body(j, carry):
            b = s + NS * j
            base = c * (E // NC) + b * EB
            pltpu.sync_copy(dst_hbm.at[pl.ds(base, EB)], dst_v)
            pltpu.sync_copy(rows_v, cnt_sh.at[dst_v], add=True)
            return carry

        lax.fori_loop(0, nblk, blk_body, 0)
        plsc.subcore_barrier()

        r0 = s * ROWS_PER_TILE
        pltpu.sync_copy(
            cnt_sh.at[pl.ds(r0, ROWS_PER_TILE), :],
            cnt_out.at[c, pl.ds(r0, ROWS_PER_TILE), :],
        )

    return pl.kernel(
        body,
        out_type=[jax.ShapeDtypeStruct((NC, NP, D), _f32)],
        mesh=_sc_mesh(),
        scratch_types=scratch,
        compiler_params=pltpu.CompilerParams(needs_layout_passes=False),
    )


def _tc_mlp_body(h_ref, p0_ref, p1_ref, c0_ref, c1_ref, scale_ref,
                 w1_ref, b1_ref, w2_ref, b2_ref, out_ref):
    cnt = c0_ref[:, 0:1] + c1_ref[:, 0:1]
    denom = jnp.maximum(cnt, 1.0)
    aggr = (p0_ref[...] + p1_ref[...]) / denom
    hmid = h_ref[...] * scale_ref[...] + aggr
    t = jnp.maximum(
        jnp.dot(hmid, w1_ref[...], preferred_element_type=_f32) + b1_ref[...], 0.0
    )
    o = jnp.dot(t, w2_ref[...], preferred_element_type=_f32) + b2_ref[...]
    out_ref[...] = jnp.maximum(o, 0.0)


def _tc_mlp(h, p0, p1, c0, c1, scale, w1, b1, w2, b2):
    R = 2000
    nb = N // R
    return pl.pallas_call(
        _tc_mlp_body,
        grid=(nb,),
        in_specs=[
            pl.BlockSpec((R, D), lambda i: (i, 0)),
            pl.BlockSpec((R, D), lambda i: (i, 0)),
            pl.BlockSpec((R, D), lambda i: (i, 0)),
            pl.BlockSpec((R, LANES), lambda i: (i, 0)),
            pl.BlockSpec((R, LANES), lambda i: (i, 0)),
            pl.BlockSpec((1, D), lambda i: (0, 0)),
            pl.BlockSpec((D, 2 * D), lambda i: (0, 0)),
            pl.BlockSpec((1, 2 * D), lambda i: (0, 0)),
            pl.BlockSpec((2 * D, D), lambda i: (0, 0)),
            pl.BlockSpec((1, D), lambda i: (0, 0)),
        ],
        out_specs=pl.BlockSpec((R, D), lambda i: (i, 0)),
        out_shape=jax.ShapeDtypeStruct((N, D), _f32),
    )(h, p0, p1, c0, c1, scale, w1, b1, w2, b2)


_sc_aggr = _make_sc_aggr()
_sc_cnt = _make_sc_cnt()


def kernel(x, edge_index, edge_attr, batch, We, be, W1, b1, W2, b2, eps):
    src = edge_index[0]
    dst = edge_index[1]
    attr16 = jnp.pad(edge_attr, ((0, 0), (0, LANES - edge_attr.shape[1]))).reshape(-1)
    h = x
    outs = []
    (cnt,) = _sc_cnt(dst)
    c0, c1 = cnt[0, :N, :LANES], cnt[1, :N, :LANES]
    for i in range(We.shape[0]):
        (p,) = _sc_aggr(h, src, dst, attr16, We[i], be[i])
        scale = jnp.full((1, D), 1.0, _f32) + eps[i]
        h = _tc_mlp(h, p[0, :N], p[1, :N], c0, c1, scale,
                    W1[i], b1[i].reshape(1, -1), W2[i], b2[i].reshape(1, -1))
        outs.append(h)
    return jnp.concatenate(outs, axis=1)
